# baseline (speedup 1.0000x reference)
import functools

import jax
import jax.numpy as jnp
from jax.experimental import pallas as pl
from jax.experimental.pallas import tpu as pltpu

MAX_LEN = 512
INFLIGHT = 4
N_PAD = 1024


def _dma_kernel(t_ref, out_ref, u8_ref, sem, *, seq_len_q, seq_len_k, max_len,
                inflight):
    block = 8
    n_blocks = seq_len_q // block
    tv = t_ref[...]
    for p in range(block):
        u8_ref[p] = pltpu.roll(tv, N_PAD - (block - 1 - p), 0) if p < block - 1 else tv

    def mk(m):
        aligned = pl.multiple_of((max_len - block) - block * m, block)
        return pltpu.make_async_copy(
            u8_ref.at[:, pl.ds(aligned, seq_len_k), :],
            out_ref.at[pl.ds(block * m, block)],
            sem,
        )

    def body(m, carry):
        mk(m).start()

        @pl.when(m >= inflight)
        def _():
            mk(m - inflight).wait()

        return carry

    jax.lax.fori_loop(0, n_blocks, body, 0)

    def tail(m, carry):
        mk(n_blocks - inflight + m).wait()
        return carry

    jax.lax.fori_loop(0, inflight, tail, 0)


def kernel(q, k, rel_pos_enc):
    seq_len_q = q.shape[1]
    seq_len_k = k.shape[1]
    d = rel_pos_enc.shape[1]
    n = rel_pos_enc.shape[0]
    padded = jnp.pad(rel_pos_enc, ((0, N_PAD - n), (0, 0)))

    body = functools.partial(
        _dma_kernel,
        seq_len_q=seq_len_q,
        seq_len_k=seq_len_k,
        max_len=MAX_LEN,
        inflight=INFLIGHT,
    )
    return pl.pallas_call(
        body,
        in_specs=[
            pl.BlockSpec(memory_space=pltpu.MemorySpace.VMEM),
        ],
        out_specs=pl.BlockSpec(memory_space=pltpu.MemorySpace.HBM),
        out_shape=jax.ShapeDtypeStruct((seq_len_q, seq_len_k, d), rel_pos_enc.dtype),
        scratch_shapes=[
            pltpu.VMEM((8, N_PAD, d), rel_pos_enc.dtype),
            pltpu.SemaphoreType.DMA,
        ],
    )(padded)
